# baseline (speedup 1.0000x reference)
import jax
import jax.numpy as jnp
from jax.experimental import pallas as pl
from jax.experimental.pallas import tpu as pltpu

_H = 128
_BATCH_TILE = 1024


def _cfc_tile(in_ref, hx_ref, ts_ref, wa_ref, wb_ref, b_ref, out_ref):
    y = jnp.dot(in_ref[...], wa_ref[...], preferred_element_type=jnp.float32)
    y = y + jnp.dot(hx_ref[...], wb_ref[...], preferred_element_type=jnp.float32)
    y = y + b_ref[...]
    ff1 = jnp.tanh(y[:, :_H])
    ff2 = jnp.tanh(y[:, _H:2 * _H])
    t_a = y[:, 2 * _H:3 * _H]
    t_b = y[:, 3 * _H:]
    t = jax.nn.sigmoid(t_a * ts_ref[...] + t_b)
    out_ref[...] = ff1 + t * (ff2 - ff1)


def kernel(input, hx, ts, W_ff1, b_ff1, W_ff2, b_ff2, W_ta, b_ta, W_tb, b_tb):
    batch, in_size = input.shape
    hid = hx.shape[1]
    cat = in_size + hid
    w = jnp.concatenate([W_ff1.T, W_ff2.T, W_ta.T, W_tb.T], axis=1)
    wa = w[:in_size]
    wb = w[in_size:]
    b = jnp.concatenate([b_ff1, b_ff2, b_ta, b_tb]).reshape(1, 4 * hid)

    tb = min(_BATCH_TILE, batch)
    grid = (batch // tb,)
    out = pl.pallas_call(
        _cfc_tile,
        grid=grid,
        in_specs=[
            pl.BlockSpec((tb, in_size), lambda i: (i, 0)),
            pl.BlockSpec((tb, hid), lambda i: (i, 0)),
            pl.BlockSpec((tb, 1), lambda i: (i, 0)),
            pl.BlockSpec((in_size, 4 * hid), lambda i: (0, 0)),
            pl.BlockSpec((hid, 4 * hid), lambda i: (0, 0)),
            pl.BlockSpec((1, 4 * hid), lambda i: (0, 0)),
        ],
        out_specs=pl.BlockSpec((tb, hid), lambda i: (i, 0)),
        out_shape=jax.ShapeDtypeStruct((batch, hid), jnp.float32),
        compiler_params=pltpu.CompilerParams(
            dimension_semantics=("parallel",),
        ),
    )(input, hx, ts, wa, wb, b)
    return (out, out)


# speedup vs baseline: 1.1074x; 1.1074x over previous
import jax
import jax.numpy as jnp
from jax.experimental import pallas as pl
from jax.experimental.pallas import tpu as pltpu

_H = 128
_BATCH_TILE = 2048


def _cfc_tile(in_ref, hx_ref, ts_ref, wa_ref, wb_ref, b_ref, out_ref):
    y = jnp.dot(in_ref[...], wa_ref[...], preferred_element_type=jnp.float32)
    y = y + jnp.dot(hx_ref[...], wb_ref[...], preferred_element_type=jnp.float32)
    y = y + b_ref[...]
    ff1 = jnp.tanh(y[:, :_H])
    ff2 = jnp.tanh(y[:, _H:2 * _H])
    t_a = y[:, 2 * _H:3 * _H]
    t_b = y[:, 3 * _H:]
    t = jax.nn.sigmoid(t_a * ts_ref[...] + t_b)
    out_ref[...] = ff1 + t * (ff2 - ff1)


def kernel(input, hx, ts, W_ff1, b_ff1, W_ff2, b_ff2, W_ta, b_ta, W_tb, b_tb):
    batch, in_size = input.shape
    hid = hx.shape[1]
    cat = in_size + hid
    w = jnp.concatenate([W_ff1.T, W_ff2.T, W_ta.T, W_tb.T], axis=1)
    wa = w[:in_size]
    wb = w[in_size:]
    b = jnp.concatenate([b_ff1, b_ff2, b_ta, b_tb]).reshape(1, 4 * hid)

    tb = min(_BATCH_TILE, batch)
    grid = (batch // tb,)
    out = pl.pallas_call(
        _cfc_tile,
        grid=grid,
        in_specs=[
            pl.BlockSpec((tb, in_size), lambda i: (i, 0)),
            pl.BlockSpec((tb, hid), lambda i: (i, 0)),
            pl.BlockSpec((tb, 1), lambda i: (i, 0)),
            pl.BlockSpec((in_size, 4 * hid), lambda i: (0, 0)),
            pl.BlockSpec((hid, 4 * hid), lambda i: (0, 0)),
            pl.BlockSpec((1, 4 * hid), lambda i: (0, 0)),
        ],
        out_specs=pl.BlockSpec((tb, hid), lambda i: (i, 0)),
        out_shape=jax.ShapeDtypeStruct((batch, hid), jnp.float32),
        compiler_params=pltpu.CompilerParams(
            dimension_semantics=("parallel",),
        ),
    )(input, hx, ts, wa, wb, b)
    return (out, out)


# speedup vs baseline: 1.1382x; 1.0279x over previous
import jax
import jax.numpy as jnp
from jax.experimental import pallas as pl
from jax.experimental.pallas import tpu as pltpu

_H = 128
_BATCH_TILE = 4096


def _cfc_tile(in_ref, hx_ref, ts_ref, wa_ref, wb_ref, b_ref, out_ref):
    y = jnp.dot(in_ref[...], wa_ref[...], preferred_element_type=jnp.float32)
    y = y + jnp.dot(hx_ref[...], wb_ref[...], preferred_element_type=jnp.float32)
    y = y + b_ref[...]
    ff1 = jnp.tanh(y[:, :_H])
    ff2 = jnp.tanh(y[:, _H:2 * _H])
    t_a = y[:, 2 * _H:3 * _H]
    t_b = y[:, 3 * _H:]
    t = jax.nn.sigmoid(t_a * ts_ref[...] + t_b)
    out_ref[...] = ff1 + t * (ff2 - ff1)


def kernel(input, hx, ts, W_ff1, b_ff1, W_ff2, b_ff2, W_ta, b_ta, W_tb, b_tb):
    batch, in_size = input.shape
    hid = hx.shape[1]
    cat = in_size + hid
    w = jnp.concatenate([W_ff1.T, W_ff2.T, W_ta.T, W_tb.T], axis=1)
    wa = w[:in_size]
    wb = w[in_size:]
    b = jnp.concatenate([b_ff1, b_ff2, b_ta, b_tb]).reshape(1, 4 * hid)

    tb = min(_BATCH_TILE, batch)
    grid = (batch // tb,)
    out = pl.pallas_call(
        _cfc_tile,
        grid=grid,
        in_specs=[
            pl.BlockSpec((tb, in_size), lambda i: (i, 0)),
            pl.BlockSpec((tb, hid), lambda i: (i, 0)),
            pl.BlockSpec((tb, 1), lambda i: (i, 0)),
            pl.BlockSpec((in_size, 4 * hid), lambda i: (0, 0)),
            pl.BlockSpec((hid, 4 * hid), lambda i: (0, 0)),
            pl.BlockSpec((1, 4 * hid), lambda i: (0, 0)),
        ],
        out_specs=pl.BlockSpec((tb, hid), lambda i: (i, 0)),
        out_shape=jax.ShapeDtypeStruct((batch, hid), jnp.float32),
        compiler_params=pltpu.CompilerParams(
            dimension_semantics=("parallel",),
        ),
    )(input, hx, ts, wa, wb, b)
    return (out, out)
